# Initial kernel scaffold; baseline (speedup 1.0000x reference)
import jax
import jax.numpy as jnp
from jax.experimental import pallas as pl


def kernel(x, edge_index, edge_attr, conv0_W, conv0_b, conv1a_W, conv1a_b, conv1b_W, conv1b_b, conv2_W, conv2_b, em_W0, em_b0, em_ln0_g, em_ln0_b, em_W1a, em_b1a, em_W1b, em_b1b, em_ln1_g, em_ln1_b, em_W2, em_b2):
    raise NotImplementedError("write your pallas kernel here")


# speedup vs baseline: 5.8410x; 5.8410x over previous
import functools

import jax
import jax.numpy as jnp
from jax import lax
from jax.experimental import pallas as pl
from jax.experimental.pallas import tpu as pltpu
from jax.experimental.pallas import tpu_sc as plsc

_N = 10000
_E = 320000
_PN = 10240
_NC = 2
_NS = 16
_NW = _NC * _NS
_EPT = _E // _NW
_C = 400
_NCH = _EPT // _C
_RPT = _PN // _NS

_BE = 6400
_G = _E // _BE


_SC_PARAMS = pltpu.CompilerParams(needs_layout_passes=False,
                                  use_tc_tiling_on_sc=False)


def _sc_mesh():
    return plsc.VectorSubcoreMesh(core_axis_name="c", subcore_axis_name="s")


def _hop(g, row, col, ew, znd, D):
    @functools.partial(
        pl.kernel,
        out_type=jax.ShapeDtypeStruct((_NC, _PN, D), jnp.float32),
        mesh=_sc_mesh(),
        scratch_types=[
            pltpu.VMEM((_C,), jnp.int32),
            pltpu.VMEM((_C,), jnp.int32),
            pltpu.VMEM((_C,), jnp.float32),
            pltpu.VMEM((_C, D), jnp.float32),
            pltpu.VMEM_SHARED((_PN, D), jnp.float32),
            pltpu.SemaphoreType.DMA,
        ],
        compiler_params=_SC_PARAMS,
    )
    def k(g_h, row_h, col_h, ew_h, z_h, out_h, ridx, cidx, wbuf, rows,
          acc_sh, sem):
        cid = lax.axis_index("c")
        sid = lax.axis_index("s")
        wid = sid * _NC + cid
        r0 = sid * _RPT
        pltpu.sync_copy(z_h.at[pl.ds(r0, _RPT)], acc_sh.at[pl.ds(r0, _RPT)])
        plsc.subcore_barrier()

        def chunk(i, carry):
            base = wid * _EPT + i * _C
            pltpu.sync_copy(row_h.at[pl.ds(base, _C)], ridx)
            pltpu.sync_copy(col_h.at[pl.ds(base, _C)], cidx)
            pltpu.sync_copy(ew_h.at[pl.ds(base, _C)], wbuf)
            pltpu.async_copy(g_h.at[ridx], rows, sem).wait()

            def scale(e, c2):
                w = plsc.load_gather(wbuf, [jnp.full((16,), e, jnp.int32)])
                for j in range(D // 16):
                    rows[e, pl.ds(j * 16, 16)] = rows[e, pl.ds(j * 16, 16)] * w
                return c2

            lax.fori_loop(0, _C, scale, 0)
            pltpu.sync_copy(rows, acc_sh.at[cidx], add=True)
            return carry

        lax.fori_loop(0, _NCH, chunk, 0)
        plsc.subcore_barrier()
        pltpu.sync_copy(acc_sh.at[pl.ds(r0, _RPT)],
                        out_h.at[cid, pl.ds(r0, _RPT)])

    return k(g, row, col, ew, znd)


def _pair_gather(A, B, row, col):
    @functools.partial(
        pl.kernel,
        out_type=jax.ShapeDtypeStruct((_E, 64), jnp.float32),
        mesh=_sc_mesh(),
        scratch_types=[
            pltpu.VMEM((_C,), jnp.int32),
            pltpu.VMEM((_C,), jnp.int32),
            pltpu.VMEM((_C, 64), jnp.float32),
            pltpu.VMEM((_C, 64), jnp.float32),
            pltpu.SemaphoreType.DMA,
            pltpu.SemaphoreType.DMA,
        ],
        compiler_params=_SC_PARAMS,
    )
    def k(a_h, b_h, row_h, col_h, out_h, ridx, cidx, bufa, bufb, sema, semb):
        cid = lax.axis_index("c")
        sid = lax.axis_index("s")
        wid = sid * _NC + cid

        def chunk(i, carry):
            base = wid * _EPT + i * _C
            pltpu.sync_copy(row_h.at[pl.ds(base, _C)], ridx)
            pltpu.sync_copy(col_h.at[pl.ds(base, _C)], cidx)
            cpa = pltpu.async_copy(a_h.at[ridx], bufa, sema)
            cpb = pltpu.async_copy(b_h.at[cidx], bufb, semb)
            cpa.wait()
            cpb.wait()

            def addrow(e, c2):
                for j in range(4):
                    bufa[e, pl.ds(j * 16, 16)] = (
                        bufa[e, pl.ds(j * 16, 16)] + bufb[e, pl.ds(j * 16, 16)])
                return c2

            lax.fori_loop(0, _C, addrow, 0)
            pltpu.sync_copy(bufa, out_h.at[pl.ds(base, _C)])
            return carry

        lax.fori_loop(0, _NCH, chunk, 0)

    return k(A, B, row, col)


def _node_mask():
    ids = lax.broadcasted_iota(jnp.int32, (_PN, 1), 0)
    return (ids < _N).astype(jnp.float32)


def _dinv_g0(degp, xp):
    def f(degp_ref, x_ref, dinv_ref, g0_ref):
        deg = (degp_ref[0] + degp_ref[1])[:, 0:1]
        safe = jnp.where(deg > 0, deg, 1.0)
        dv = jnp.where(deg > 0, lax.rsqrt(safe), 0.0)
        dinv_ref[...] = dv
        g0_ref[...] = x_ref[...] * dv

    return pl.pallas_call(
        f,
        out_shape=(jax.ShapeDtypeStruct((_PN, 1), jnp.float32),
                   jax.ShapeDtypeStruct((_PN, 128), jnp.float32)),
    )(degp, xp)


def _mid(t, p1, dinv, W0, W1, D):
    def f(t_ref, p_ref, dv_ref, w0_ref, w1_ref, g2_ref, acc_ref):
        dv = dv_ref[...]
        h1 = (p_ref[0] + p_ref[1]) * dv
        g2_ref[...] = h1 * dv
        acc_ref[...] = (
            jnp.dot(t_ref[...], w0_ref[...], preferred_element_type=jnp.float32)
            + jnp.dot(h1, w1_ref[...], preferred_element_type=jnp.float32))

    return pl.pallas_call(
        f,
        out_shape=(jax.ShapeDtypeStruct((_PN, D), jnp.float32),
                   jax.ShapeDtypeStruct((_PN, 64), jnp.float32)),
    )(t, p1, dinv, W0, W1)


def _final(acc, p2, dinv, W2, b, res, do_instnorm, emW=None):
    nouts = 4 if emW is not None else 2
    has_res = res is not None

    def f(*refs):
        i = 0
        acc_ref = refs[i]; i += 1
        p_ref = refs[i]; i += 1
        dv_ref = refs[i]; i += 1
        w2_ref = refs[i]; i += 1
        b_ref = refs[i]; i += 1
        res_ref = None
        if has_res:
            res_ref = refs[i]; i += 1
        wa_ref = wb_ref = None
        if emW is not None:
            wa_ref = refs[i]; i += 1
            wb_ref = refs[i]; i += 1
        outs = refs[i:]

        dv = dv_ref[...]
        h2 = (p_ref[0] + p_ref[1]) * dv
        d = acc_ref[...] + jnp.dot(h2, w2_ref[...],
                                   preferred_element_type=jnp.float32)
        d = d + b_ref[...]
        mask = _node_mask()
        if do_instnorm:
            m = jnp.sum(d * mask, axis=0, keepdims=True) / _N
            c = d - m
            v = jnp.sum(c * c * mask, axis=0, keepdims=True) / _N
            d = c * lax.rsqrt(v + 1e-5)
        if has_res:
            d = d + res_ref[...]
        data = jnp.maximum(d, 0.0) * mask
        outs[0][...] = data
        outs[1][...] = data * dv
        if emW is not None:
            outs[2][...] = jnp.dot(data, wa_ref[...],
                                   preferred_element_type=jnp.float32)
            outs[3][...] = jnp.dot(data, wb_ref[...],
                                   preferred_element_type=jnp.float32)

    out_shape = tuple(jax.ShapeDtypeStruct((_PN, 64), jnp.float32)
                      for _ in range(nouts))
    args = [acc, p2, dinv, W2, b]
    if has_res:
        args.append(res)
    if emW is not None:
        args.extend(emW)
    return pl.pallas_call(f, out_shape=out_shape)(*args)


def _edge_mlp(y0, ea, wc, b0, g0, be0, W1a, b1a, W1b, b1b, g1, be1, w2, b2):
    def ln(y, g, b):
        m = jnp.mean(y, axis=-1, keepdims=True)
        v = jnp.mean((y - m) ** 2, axis=-1, keepdims=True)
        return (y - m) * lax.rsqrt(v + 1e-5) * g + b

    def f(y_ref, ea_ref, wc_ref, b0_ref, g0_ref, be0_ref, w1a_ref, b1a_ref,
          w1b_ref, b1b_ref, g1_ref, be1_ref, w2_ref, b2_ref, out_ref):
        ew = ea_ref[...]
        y = y_ref[...] + ew * wc_ref[...] + b0_ref[...]
        y = ln(y, g0_ref[...], be0_ref[...])
        h = jnp.maximum(y, 0.0)
        r = h
        y = jnp.maximum(
            jnp.dot(h, w1a_ref[...], preferred_element_type=jnp.float32)
            + b1a_ref[...], 0.0)
        y = jnp.dot(y, w1b_ref[...], preferred_element_type=jnp.float32)
        y = y + b1b_ref[...]
        y = ln(y, g1_ref[...], be1_ref[...]) + r
        h = jnp.maximum(y, 0.0)
        e = jnp.sum(h * w2_ref[...], axis=1, keepdims=True) + b2_ref[...]
        out_ref[...] = e.reshape(1, _BE // 128, 128)

    full = lambda s: pl.BlockSpec(s, lambda i: (0,) * len(s))
    return pl.pallas_call(
        f,
        grid=(_G,),
        in_specs=[
            pl.BlockSpec((_BE, 64), lambda i: (i, 0)),
            pl.BlockSpec((_BE, 1), lambda i: (i, 0)),
            full((1, 64)), full((1, 64)), full((1, 64)), full((1, 64)),
            full((64, 64)), full((1, 64)), full((64, 64)), full((1, 64)),
            full((1, 64)), full((1, 64)), full((1, 64)), full((1, 1)),
        ],
        out_specs=pl.BlockSpec((1, _BE // 128, 128), lambda i: (i, 0, 0)),
        out_shape=jax.ShapeDtypeStruct((_G, _BE // 128, 128), jnp.float32),
    )(y0, ea, wc, b0, g0, be0, W1a, b1a, W1b, b1b, g1, be1, w2, b2)


def _standardize(ep):
    def f(e_ref, out_ref):
        e = e_ref[...]
        m = jnp.sum(e) / _E
        c = e - m
        sd = jnp.sqrt(jnp.sum(c * c) / (_E - 1))
        out_ref[...] = jnp.abs(c / sd)

    return pl.pallas_call(
        f, out_shape=jax.ShapeDtypeStruct((_G, _BE // 128, 128),
                                          jnp.float32))(ep)


def kernel(x, edge_index, edge_attr, conv0_W, conv0_b, conv1a_W, conv1a_b,
           conv1b_W, conv1b_b, conv2_W, conv2_b, em_W0, em_b0, em_ln0_g,
           em_ln0_b, em_W1a, em_b1a, em_W1b, em_b1b, em_ln1_g, em_ln1_b,
           em_W2, em_b2):
    row = edge_index[0]
    col = edge_index[1]
    ew = edge_attr.reshape(-1)
    xp = jnp.pad(x, ((0, _PN - _N), (0, 0)))
    z16 = jnp.zeros((_PN, 16), jnp.float32)
    z64 = jnp.zeros((_PN, 64), jnp.float32)
    row1 = lambda a: a.reshape(1, -1)

    degp = _hop(jnp.ones((_PN, 16), jnp.float32), col, col, ew, z16, 16)
    dinv, g0 = _dinv_g0(degp, xp)

    def layer(t, g, D, W, b, res, do_in, emW=None):
        if D == 128:
            hopf = lambda gg: jnp.concatenate(
                [_hop(gg[:, :64], row, col, ew, z64, 64),
                 _hop(gg[:, 64:], row, col, ew, z64, 64)], axis=2)
        else:
            hopf = lambda gg: _hop(gg, row, col, ew, z64, 64)
        p1 = hopf(g)
        g2, acc = _mid(t, p1, dinv, W[0], W[1], D)
        p2 = hopf(g2)
        return _final(acc, p2, dinv, W[2], row1(b), res, do_in, emW)

    data0, gn0 = layer(xp, g0, 128, conv0_W, conv0_b, None, True)
    data1, gn1 = layer(data0, gn0, 64, conv1a_W, conv1a_b, None, False)
    data2, gn2 = layer(data1, gn1, 64, conv1b_W, conv1b_b, data0, True)
    _, _, A, B = layer(data2, gn2, 64, conv2_W, conv2_b, data2, False,
                       emW=(em_W0[:64], em_W0[64:128]))

    y0 = _pair_gather(A, B, row, col)
    ep = _edge_mlp(y0, edge_attr, row1(em_W0[128]), row1(em_b0),
                   row1(em_ln0_g), row1(em_ln0_b), em_W1a, row1(em_b1a),
                   em_W1b, row1(em_b1b), row1(em_ln1_g), row1(em_ln1_b),
                   row1(em_W2.reshape(-1)), em_b2.reshape(1, 1))
    e = _standardize(ep)
    return e.reshape(_E, 1)


# speedup vs baseline: 6.6170x; 1.1329x over previous
import functools

import jax
import jax.numpy as jnp
from jax import lax
from jax.experimental import pallas as pl
from jax.experimental.pallas import tpu as pltpu
from jax.experimental.pallas import tpu_sc as plsc

_N = 10000
_E = 320000
_PN = 10240
_NC = 2
_NS = 16
_NW = _NC * _NS
_EPT = _E // _NW
_C = 400
_NCH = _EPT // _C
_RPT = _PN // _NS

_BE = 6400
_G = _E // _BE


_SC_PARAMS = pltpu.CompilerParams(needs_layout_passes=False,
                                  use_tc_tiling_on_sc=False)


def _sc_mesh():
    return plsc.VectorSubcoreMesh(core_axis_name="c", subcore_axis_name="s")


def _hop(g, row, col, ew, znd, D):
    @functools.partial(
        pl.kernel,
        out_type=jax.ShapeDtypeStruct((_NC, _PN, D), jnp.float32),
        mesh=_sc_mesh(),
        scratch_types=[
            pltpu.VMEM((_C,), jnp.int32),
            pltpu.VMEM((_C,), jnp.int32),
            pltpu.VMEM((_C,), jnp.float32),
            pltpu.VMEM((_C, D), jnp.float32),
            pltpu.VMEM_SHARED((_PN, D), jnp.float32),
            pltpu.SemaphoreType.DMA,
        ],
        compiler_params=_SC_PARAMS,
    )
    def k(g_h, row_h, col_h, ew_h, z_h, out_h, ridx, cidx, wbuf, rows,
          acc_sh, sem):
        cid = lax.axis_index("c")
        sid = lax.axis_index("s")
        wid = sid * _NC + cid
        r0 = sid * _RPT
        pltpu.sync_copy(z_h.at[pl.ds(r0, _RPT)], acc_sh.at[pl.ds(r0, _RPT)])
        plsc.subcore_barrier()

        def chunk(i, carry):
            base = wid * _EPT + i * _C
            pltpu.sync_copy(row_h.at[pl.ds(base, _C)], ridx)
            pltpu.sync_copy(col_h.at[pl.ds(base, _C)], cidx)
            pltpu.sync_copy(ew_h.at[pl.ds(base, _C)], wbuf)
            pltpu.async_copy(g_h.at[ridx], rows, sem).wait()

            def scale16(q, c2):
                w16 = wbuf[pl.ds(q * 16, 16)]
                for l in range(16):
                    wb = jnp.broadcast_to(w16[l], (16,))
                    e = q * 16 + l
                    for j in range(D // 16):
                        rows[e, pl.ds(j * 16, 16)] = (
                            rows[e, pl.ds(j * 16, 16)] * wb)
                return c2

            lax.fori_loop(0, _C // 16, scale16, 0)
            pltpu.sync_copy(rows, acc_sh.at[cidx], add=True)
            return carry

        lax.fori_loop(0, _NCH, chunk, 0)
        plsc.subcore_barrier()
        pltpu.sync_copy(acc_sh.at[pl.ds(r0, _RPT)],
                        out_h.at[cid, pl.ds(r0, _RPT)])

    return k(g, row, col, ew, znd)


def _pair_gather(A, B, row, col):
    @functools.partial(
        pl.kernel,
        out_type=jax.ShapeDtypeStruct((_E, 64), jnp.float32),
        mesh=_sc_mesh(),
        scratch_types=[
            pltpu.VMEM((_C,), jnp.int32),
            pltpu.VMEM((_C,), jnp.int32),
            pltpu.VMEM((_C, 64), jnp.float32),
            pltpu.VMEM((_C, 64), jnp.float32),
            pltpu.SemaphoreType.DMA,
            pltpu.SemaphoreType.DMA,
        ],
        compiler_params=_SC_PARAMS,
    )
    def k(a_h, b_h, row_h, col_h, out_h, ridx, cidx, bufa, bufb, sema, semb):
        cid = lax.axis_index("c")
        sid = lax.axis_index("s")
        wid = sid * _NC + cid

        def chunk(i, carry):
            base = wid * _EPT + i * _C
            pltpu.sync_copy(row_h.at[pl.ds(base, _C)], ridx)
            pltpu.sync_copy(col_h.at[pl.ds(base, _C)], cidx)
            cpa = pltpu.async_copy(a_h.at[ridx], bufa, sema)
            cpb = pltpu.async_copy(b_h.at[cidx], bufb, semb)
            cpa.wait()
            cpb.wait()

            def addrow(e, c2):
                for j in range(4):
                    bufa[e, pl.ds(j * 16, 16)] = (
                        bufa[e, pl.ds(j * 16, 16)] + bufb[e, pl.ds(j * 16, 16)])
                return c2

            lax.fori_loop(0, _C, addrow, 0)
            pltpu.sync_copy(bufa, out_h.at[pl.ds(base, _C)])
            return carry

        lax.fori_loop(0, _NCH, chunk, 0)

    return k(A, B, row, col)


def _node_mask():
    ids = lax.broadcasted_iota(jnp.int32, (_PN, 1), 0)
    return (ids < _N).astype(jnp.float32)


def _dinv_g0(degp, xp):
    def f(degp_ref, x_ref, dinv_ref, g0_ref):
        deg = (degp_ref[0] + degp_ref[1])[:, 0:1]
        safe = jnp.where(deg > 0, deg, 1.0)
        dv = jnp.where(deg > 0, lax.rsqrt(safe), 0.0)
        dinv_ref[...] = dv
        g0_ref[...] = x_ref[...] * dv

    return pl.pallas_call(
        f,
        out_shape=(jax.ShapeDtypeStruct((_PN, 1), jnp.float32),
                   jax.ShapeDtypeStruct((_PN, 128), jnp.float32)),
    )(degp, xp)


def _mid(t, p1, dinv, W0, W1, D):
    def f(t_ref, p_ref, dv_ref, w0_ref, w1_ref, g2_ref, acc_ref):
        dv = dv_ref[...]
        h1 = (p_ref[0] + p_ref[1]) * dv
        g2_ref[...] = h1 * dv
        acc_ref[...] = (
            jnp.dot(t_ref[...], w0_ref[...], preferred_element_type=jnp.float32)
            + jnp.dot(h1, w1_ref[...], preferred_element_type=jnp.float32))

    return pl.pallas_call(
        f,
        out_shape=(jax.ShapeDtypeStruct((_PN, D), jnp.float32),
                   jax.ShapeDtypeStruct((_PN, 64), jnp.float32)),
    )(t, p1, dinv, W0, W1)


def _final(acc, p2, dinv, W2, b, res, do_instnorm, emW=None):
    nouts = 4 if emW is not None else 2
    has_res = res is not None

    def f(*refs):
        i = 0
        acc_ref = refs[i]; i += 1
        p_ref = refs[i]; i += 1
        dv_ref = refs[i]; i += 1
        w2_ref = refs[i]; i += 1
        b_ref = refs[i]; i += 1
        res_ref = None
        if has_res:
            res_ref = refs[i]; i += 1
        wa_ref = wb_ref = None
        if emW is not None:
            wa_ref = refs[i]; i += 1
            wb_ref = refs[i]; i += 1
        outs = refs[i:]

        dv = dv_ref[...]
        h2 = (p_ref[0] + p_ref[1]) * dv
        d = acc_ref[...] + jnp.dot(h2, w2_ref[...],
                                   preferred_element_type=jnp.float32)
        d = d + b_ref[...]
        mask = _node_mask()
        if do_instnorm:
            m = jnp.sum(d * mask, axis=0, keepdims=True) / _N
            c = d - m
            v = jnp.sum(c * c * mask, axis=0, keepdims=True) / _N
            d = c * lax.rsqrt(v + 1e-5)
        if has_res:
            d = d + res_ref[...]
        data = jnp.maximum(d, 0.0) * mask
        outs[0][...] = data
        outs[1][...] = data * dv
        if emW is not None:
            outs[2][...] = jnp.dot(data, wa_ref[...],
                                   preferred_element_type=jnp.float32)
            outs[3][...] = jnp.dot(data, wb_ref[...],
                                   preferred_element_type=jnp.float32)

    out_shape = tuple(jax.ShapeDtypeStruct((_PN, 64), jnp.float32)
                      for _ in range(nouts))
    args = [acc, p2, dinv, W2, b]
    if has_res:
        args.append(res)
    if emW is not None:
        args.extend(emW)
    return pl.pallas_call(f, out_shape=out_shape)(*args)


def _edge_mlp(y0, ea, wc, b0, g0, be0, W1a, b1a, W1b, b1b, g1, be1, w2, b2):
    def ln(y, g, b):
        m = jnp.mean(y, axis=-1, keepdims=True)
        v = jnp.mean((y - m) ** 2, axis=-1, keepdims=True)
        return (y - m) * lax.rsqrt(v + 1e-5) * g + b

    def f(y_ref, ea_ref, wc_ref, b0_ref, g0_ref, be0_ref, w1a_ref, b1a_ref,
          w1b_ref, b1b_ref, g1_ref, be1_ref, w2_ref, b2_ref, out_ref):
        ew = ea_ref[...]
        y = y_ref[...] + ew * wc_ref[...] + b0_ref[...]
        y = ln(y, g0_ref[...], be0_ref[...])
        h = jnp.maximum(y, 0.0)
        r = h
        y = jnp.maximum(
            jnp.dot(h, w1a_ref[...], preferred_element_type=jnp.float32)
            + b1a_ref[...], 0.0)
        y = jnp.dot(y, w1b_ref[...], preferred_element_type=jnp.float32)
        y = y + b1b_ref[...]
        y = ln(y, g1_ref[...], be1_ref[...]) + r
        h = jnp.maximum(y, 0.0)
        e = jnp.sum(h * w2_ref[...], axis=1, keepdims=True) + b2_ref[...]
        out_ref[...] = e.reshape(1, _BE // 128, 128)

    full = lambda s: pl.BlockSpec(s, lambda i: (0,) * len(s))
    return pl.pallas_call(
        f,
        grid=(_G,),
        in_specs=[
            pl.BlockSpec((_BE, 64), lambda i: (i, 0)),
            pl.BlockSpec((_BE, 1), lambda i: (i, 0)),
            full((1, 64)), full((1, 64)), full((1, 64)), full((1, 64)),
            full((64, 64)), full((1, 64)), full((64, 64)), full((1, 64)),
            full((1, 64)), full((1, 64)), full((1, 64)), full((1, 1)),
        ],
        out_specs=pl.BlockSpec((1, _BE // 128, 128), lambda i: (i, 0, 0)),
        out_shape=jax.ShapeDtypeStruct((_G, _BE // 128, 128), jnp.float32),
    )(y0, ea, wc, b0, g0, be0, W1a, b1a, W1b, b1b, g1, be1, w2, b2)


def _standardize(ep):
    def f(e_ref, out_ref):
        e = e_ref[...]
        m = jnp.sum(e) / _E
        c = e - m
        sd = jnp.sqrt(jnp.sum(c * c) / (_E - 1))
        out_ref[...] = jnp.abs(c / sd)

    return pl.pallas_call(
        f, out_shape=jax.ShapeDtypeStruct((_G, _BE // 128, 128),
                                          jnp.float32))(ep)


def kernel(x, edge_index, edge_attr, conv0_W, conv0_b, conv1a_W, conv1a_b,
           conv1b_W, conv1b_b, conv2_W, conv2_b, em_W0, em_b0, em_ln0_g,
           em_ln0_b, em_W1a, em_b1a, em_W1b, em_b1b, em_ln1_g, em_ln1_b,
           em_W2, em_b2):
    row = edge_index[0]
    col = edge_index[1]
    ew = edge_attr.reshape(-1)
    xp = jnp.pad(x, ((0, _PN - _N), (0, 0)))
    z16 = jnp.zeros((_PN, 16), jnp.float32)
    z64 = jnp.zeros((_PN, 64), jnp.float32)
    row1 = lambda a: a.reshape(1, -1)

    degp = _hop(jnp.ones((_PN, 16), jnp.float32), col, col, ew, z16, 16)
    dinv, g0 = _dinv_g0(degp, xp)

    def layer(t, g, D, W, b, res, do_in, emW=None):
        if D == 128:
            hopf = lambda gg: jnp.concatenate(
                [_hop(gg[:, :64], row, col, ew, z64, 64),
                 _hop(gg[:, 64:], row, col, ew, z64, 64)], axis=2)
        else:
            hopf = lambda gg: _hop(gg, row, col, ew, z64, 64)
        p1 = hopf(g)
        g2, acc = _mid(t, p1, dinv, W[0], W[1], D)
        p2 = hopf(g2)
        return _final(acc, p2, dinv, W[2], row1(b), res, do_in, emW)

    data0, gn0 = layer(xp, g0, 128, conv0_W, conv0_b, None, True)
    data1, gn1 = layer(data0, gn0, 64, conv1a_W, conv1a_b, None, False)
    data2, gn2 = layer(data1, gn1, 64, conv1b_W, conv1b_b, data0, True)
    _, _, A, B = layer(data2, gn2, 64, conv2_W, conv2_b, data2, False,
                       emW=(em_W0[:64], em_W0[64:128]))

    y0 = _pair_gather(A, B, row, col)
    ep = _edge_mlp(y0, edge_attr, row1(em_W0[128]), row1(em_b0),
                   row1(em_ln0_g), row1(em_ln0_b), em_W1a, row1(em_b1a),
                   em_W1b, row1(em_b1b), row1(em_ln1_g), row1(em_ln1_b),
                   row1(em_W2.reshape(-1)), em_b2.reshape(1, 1))
    e = _standardize(ep)
    return e.reshape(_E, 1)


# speedup vs baseline: 7.6417x; 1.1549x over previous
import functools

import jax
import jax.numpy as jnp
from jax import lax
from jax.experimental import pallas as pl
from jax.experimental.pallas import tpu as pltpu
from jax.experimental.pallas import tpu_sc as plsc

_N = 10000
_E = 320000
_PN = 10240
_NC = 2
_NS = 16
_NW = _NC * _NS
_EPT = _E // _NW
_C = 400
_NCH = _EPT // _C
_RPT = _PN // _NS

_BE = 6400
_G = _E // _BE


_SC_PARAMS = pltpu.CompilerParams(needs_layout_passes=False,
                                  use_tc_tiling_on_sc=False)


def _sc_mesh():
    return plsc.VectorSubcoreMesh(core_axis_name="c", subcore_axis_name="s")


def _hop(g, row3, col3, ew3, znd, D):
    @functools.partial(
        pl.kernel,
        out_type=jax.ShapeDtypeStruct((_NC, _PN, D), jnp.float32),
        mesh=_sc_mesh(),
        scratch_types=[
            pltpu.VMEM((_NCH, _C), jnp.int32),
            pltpu.VMEM((_NCH, _C), jnp.int32),
            pltpu.VMEM((_NCH, _C), jnp.float32),
            pltpu.VMEM((2, _C, D), jnp.float32),
            pltpu.VMEM_SHARED((_PN, D), jnp.float32),
            pltpu.SemaphoreType.DMA,
            pltpu.SemaphoreType.DMA,
        ],
        compiler_params=_SC_PARAMS,
    )
    def k(g_h, row_h, col_h, ew_h, z_h, out_h, ridx, cidx, wbuf, rows,
          acc_sh, sem0, sem1):
        sems = (sem0, sem1)
        cid = lax.axis_index("c")
        sid = lax.axis_index("s")
        wid = sid * _NC + cid
        r0 = sid * _RPT
        pltpu.sync_copy(row_h.at[wid], ridx)
        pltpu.sync_copy(col_h.at[wid], cidx)
        pltpu.sync_copy(ew_h.at[wid], wbuf)
        pltpu.sync_copy(z_h.at[pl.ds(r0, _RPT)], acc_sh.at[pl.ds(r0, _RPT)])
        plsc.subcore_barrier()

        def scale(rb, i):
            def scale16(q, c2):
                w16 = wbuf[i, pl.ds(q * 16, 16)]
                for l in range(16):
                    wb = jnp.broadcast_to(w16[l], (16,))
                    e = q * 16 + l
                    for j in range(D // 16):
                        rb[e, pl.ds(j * 16, 16)] = rb[e, pl.ds(j * 16, 16)] * wb
                return c2

            lax.fori_loop(0, _C // 16, scale16, 0)

        def do_chunk(i, b, issue_next):
            pltpu.make_async_copy(g_h.at[ridx.at[i]], rows.at[b],
                                  sems[b]).wait()
            if issue_next:
                pltpu.async_copy(g_h.at[ridx.at[i + 1]], rows.at[1 - b],
                                 sems[1 - b])
            scale(rows.at[b], i)
            pltpu.sync_copy(rows.at[b], acc_sh.at[cidx.at[i]], add=True)

        pltpu.async_copy(g_h.at[ridx.at[0]], rows.at[0], sems[0])

        def pair(i2, carry):
            do_chunk(2 * i2, 0, True)
            do_chunk(2 * i2 + 1, 1, True)
            return carry

        lax.fori_loop(0, (_NCH - 1) // 2, pair, 0)
        do_chunk(_NCH - 1, 0, False)

        plsc.subcore_barrier()
        pltpu.sync_copy(acc_sh.at[pl.ds(r0, _RPT)],
                        out_h.at[cid, pl.ds(r0, _RPT)])

    return k(g, row3, col3, ew3, znd)


def _pair_gather(A, B, row3, col3):
    @functools.partial(
        pl.kernel,
        out_type=jax.ShapeDtypeStruct((_E, 64), jnp.float32),
        mesh=_sc_mesh(),
        scratch_types=[
            pltpu.VMEM((_NCH, _C), jnp.int32),
            pltpu.VMEM((_NCH, _C), jnp.int32),
            pltpu.VMEM((2, _C, 64), jnp.float32),
            pltpu.VMEM((2, _C, 64), jnp.float32),
            pltpu.SemaphoreType.DMA,
            pltpu.SemaphoreType.DMA,
        ],
        compiler_params=_SC_PARAMS,
    )
    def k(a_h, b_h, row_h, col_h, out_h, ridx, cidx, bufa, bufb, sem0, sem1):
        sems = (sem0, sem1)
        cid = lax.axis_index("c")
        sid = lax.axis_index("s")
        wid = sid * _NC + cid
        pltpu.sync_copy(row_h.at[wid], ridx)
        pltpu.sync_copy(col_h.at[wid], cidx)

        def issue(i, b):
            pltpu.async_copy(a_h.at[ridx.at[i]], bufa.at[b], sems[b])
            pltpu.async_copy(b_h.at[cidx.at[i]], bufb.at[b], sems[b])

        def do_chunk(i, b, issue_next):
            pltpu.make_async_copy(a_h.at[ridx.at[i]], bufa.at[b],
                                  sems[b]).wait()
            pltpu.make_async_copy(b_h.at[cidx.at[i]], bufb.at[b],
                                  sems[b]).wait()
            if issue_next:
                issue(i + 1, 1 - b)

            def addrow(e, c2):
                for j in range(4):
                    bufa.at[b][e, pl.ds(j * 16, 16)] = (
                        bufa.at[b][e, pl.ds(j * 16, 16)]
                        + bufb.at[b][e, pl.ds(j * 16, 16)])
                return c2

            lax.fori_loop(0, _C, addrow, 0)
            base = wid * _EPT + i * _C
            pltpu.sync_copy(bufa.at[b], out_h.at[pl.ds(base, _C)])

        issue(0, 0)

        def pair(i2, carry):
            do_chunk(2 * i2, 0, True)
            do_chunk(2 * i2 + 1, 1, True)
            return carry

        lax.fori_loop(0, (_NCH - 1) // 2, pair, 0)
        do_chunk(_NCH - 1, 0, False)

    return k(A, B, row3, col3)


def _node_mask():
    ids = lax.broadcasted_iota(jnp.int32, (_PN, 1), 0)
    return (ids < _N).astype(jnp.float32)


def _dinv_g0(degp, xp):
    def f(degp_ref, x_ref, dinv_ref, g0_ref):
        deg = (degp_ref[0] + degp_ref[1])[:, 0:1]
        safe = jnp.where(deg > 0, deg, 1.0)
        dv = jnp.where(deg > 0, lax.rsqrt(safe), 0.0)
        dinv_ref[...] = dv
        g0_ref[...] = x_ref[...] * dv

    return pl.pallas_call(
        f,
        out_shape=(jax.ShapeDtypeStruct((_PN, 1), jnp.float32),
                   jax.ShapeDtypeStruct((_PN, 128), jnp.float32)),
    )(degp, xp)


def _mid(t, p1, dinv, W0, W1, D):
    def f(t_ref, p_ref, dv_ref, w0_ref, w1_ref, g2_ref, acc_ref):
        dv = dv_ref[...]
        h1 = (p_ref[0] + p_ref[1]) * dv
        g2_ref[...] = h1 * dv
        acc_ref[...] = (
            jnp.dot(t_ref[...], w0_ref[...], preferred_element_type=jnp.float32)
            + jnp.dot(h1, w1_ref[...], preferred_element_type=jnp.float32))

    return pl.pallas_call(
        f,
        out_shape=(jax.ShapeDtypeStruct((_PN, D), jnp.float32),
                   jax.ShapeDtypeStruct((_PN, 64), jnp.float32)),
    )(t, p1, dinv, W0, W1)


def _final(acc, p2, dinv, W2, b, res, do_instnorm, emW=None):
    nouts = 4 if emW is not None else 2
    has_res = res is not None

    def f(*refs):
        i = 0
        acc_ref = refs[i]; i += 1
        p_ref = refs[i]; i += 1
        dv_ref = refs[i]; i += 1
        w2_ref = refs[i]; i += 1
        b_ref = refs[i]; i += 1
        res_ref = None
        if has_res:
            res_ref = refs[i]; i += 1
        wa_ref = wb_ref = None
        if emW is not None:
            wa_ref = refs[i]; i += 1
            wb_ref = refs[i]; i += 1
        outs = refs[i:]

        dv = dv_ref[...]
        h2 = (p_ref[0] + p_ref[1]) * dv
        d = acc_ref[...] + jnp.dot(h2, w2_ref[...],
                                   preferred_element_type=jnp.float32)
        d = d + b_ref[...]
        mask = _node_mask()
        if do_instnorm:
            m = jnp.sum(d * mask, axis=0, keepdims=True) / _N
            c = d - m
            v = jnp.sum(c * c * mask, axis=0, keepdims=True) / _N
            d = c * lax.rsqrt(v + 1e-5)
        if has_res:
            d = d + res_ref[...]
        data = jnp.maximum(d, 0.0) * mask
        outs[0][...] = data
        outs[1][...] = data * dv
        if emW is not None:
            outs[2][...] = jnp.dot(data, wa_ref[...],
                                   preferred_element_type=jnp.float32)
            outs[3][...] = jnp.dot(data, wb_ref[...],
                                   preferred_element_type=jnp.float32)

    out_shape = tuple(jax.ShapeDtypeStruct((_PN, 64), jnp.float32)
                      for _ in range(nouts))
    args = [acc, p2, dinv, W2, b]
    if has_res:
        args.append(res)
    if emW is not None:
        args.extend(emW)
    return pl.pallas_call(f, out_shape=out_shape)(*args)


def _edge_mlp(y0, ea, wc, b0, g0, be0, W1a, b1a, W1b, b1b, g1, be1, w2, b2):
    def ln(y, g, b):
        m = jnp.mean(y, axis=-1, keepdims=True)
        v = jnp.mean((y - m) ** 2, axis=-1, keepdims=True)
        return (y - m) * lax.rsqrt(v + 1e-5) * g + b

    def f(y_ref, ea_ref, wc_ref, b0_ref, g0_ref, be0_ref, w1a_ref, b1a_ref,
          w1b_ref, b1b_ref, g1_ref, be1_ref, w2_ref, b2_ref, out_ref):
        ew = ea_ref[...]
        y = y_ref[...] + ew * wc_ref[...] + b0_ref[...]
        y = ln(y, g0_ref[...], be0_ref[...])
        h = jnp.maximum(y, 0.0)
        r = h
        y = jnp.maximum(
            jnp.dot(h, w1a_ref[...], preferred_element_type=jnp.float32)
            + b1a_ref[...], 0.0)
        y = jnp.dot(y, w1b_ref[...], preferred_element_type=jnp.float32)
        y = y + b1b_ref[...]
        y = ln(y, g1_ref[...], be1_ref[...]) + r
        h = jnp.maximum(y, 0.0)
        e = jnp.sum(h * w2_ref[...], axis=1, keepdims=True) + b2_ref[...]
        out_ref[...] = e.reshape(1, _BE // 128, 128)

    full = lambda s: pl.BlockSpec(s, lambda i: (0,) * len(s))
    return pl.pallas_call(
        f,
        grid=(_G,),
        in_specs=[
            pl.BlockSpec((_BE, 64), lambda i: (i, 0)),
            pl.BlockSpec((_BE, 1), lambda i: (i, 0)),
            full((1, 64)), full((1, 64)), full((1, 64)), full((1, 64)),
            full((64, 64)), full((1, 64)), full((64, 64)), full((1, 64)),
            full((1, 64)), full((1, 64)), full((1, 64)), full((1, 1)),
        ],
        out_specs=pl.BlockSpec((1, _BE // 128, 128), lambda i: (i, 0, 0)),
        out_shape=jax.ShapeDtypeStruct((_G, _BE // 128, 128), jnp.float32),
    )(y0, ea, wc, b0, g0, be0, W1a, b1a, W1b, b1b, g1, be1, w2, b2)


def _standardize(ep):
    def f(e_ref, out_ref):
        e = e_ref[...]
        m = jnp.sum(e) / _E
        c = e - m
        sd = jnp.sqrt(jnp.sum(c * c) / (_E - 1))
        out_ref[...] = jnp.abs(c / sd)

    return pl.pallas_call(
        f, out_shape=jax.ShapeDtypeStruct((_G, _BE // 128, 128),
                                          jnp.float32))(ep)


def kernel(x, edge_index, edge_attr, conv0_W, conv0_b, conv1a_W, conv1a_b,
           conv1b_W, conv1b_b, conv2_W, conv2_b, em_W0, em_b0, em_ln0_g,
           em_ln0_b, em_W1a, em_b1a, em_W1b, em_b1b, em_ln1_g, em_ln1_b,
           em_W2, em_b2):
    row = edge_index[0].reshape(_NW, _NCH, _C)
    col = edge_index[1].reshape(_NW, _NCH, _C)
    ew = edge_attr.reshape(_NW, _NCH, _C)
    xp = jnp.pad(x, ((0, _PN - _N), (0, 0)))
    z16 = jnp.zeros((_PN, 16), jnp.float32)
    z64 = jnp.zeros((_PN, 64), jnp.float32)
    row1 = lambda a: a.reshape(1, -1)

    degp = _hop(jnp.ones((_PN, 16), jnp.float32), col, col, ew, z16, 16)
    dinv, g0 = _dinv_g0(degp, xp)

    def layer(t, g, D, W, b, res, do_in, emW=None):
        if D == 128:
            hopf = lambda gg: jnp.concatenate(
                [_hop(gg[:, :64], row, col, ew, z64, 64),
                 _hop(gg[:, 64:], row, col, ew, z64, 64)], axis=2)
        else:
            hopf = lambda gg: _hop(gg, row, col, ew, z64, 64)
        p1 = hopf(g)
        g2, acc = _mid(t, p1, dinv, W[0], W[1], D)
        p2 = hopf(g2)
        return _final(acc, p2, dinv, W[2], row1(b), res, do_in, emW)

    data0, gn0 = layer(xp, g0, 128, conv0_W, conv0_b, None, True)
    data1, gn1 = layer(data0, gn0, 64, conv1a_W, conv1a_b, None, False)
    data2, gn2 = layer(data1, gn1, 64, conv1b_W, conv1b_b, data0, True)
    _, _, A, B = layer(data2, gn2, 64, conv2_W, conv2_b, data2, False,
                       emW=(em_W0[:64], em_W0[64:128]))

    y0 = _pair_gather(A, B, row, col)
    ep = _edge_mlp(y0, edge_attr, row1(em_W0[128]), row1(em_b0),
                   row1(em_ln0_g), row1(em_ln0_b), em_W1a, row1(em_b1a),
                   em_W1b, row1(em_b1b), row1(em_ln1_g), row1(em_ln1_b),
                   row1(em_W2.reshape(-1)), em_b2.reshape(1, 1))
    e = _standardize(ep)
    return e.reshape(_E, 1)
